# Initial kernel scaffold; baseline (speedup 1.0000x reference)
import jax
import jax.numpy as jnp
from jax.experimental import pallas as pl


def kernel(hidden_states, weight):
    raise NotImplementedError("write your pallas kernel here")


# speedup vs baseline: 1.6239x; 1.6239x over previous
import jax
import jax.numpy as jnp
from jax.experimental import pallas as pl


def _gate_body(x_ref, wt_ref, idx_ref, w_ref):
    logits = jnp.dot(x_ref[...], wt_ref[...], preferred_element_type=jnp.float32)
    m1 = jnp.max(logits, axis=1, keepdims=True)
    i1 = jnp.argmax(logits, axis=1)[:, None].astype(jnp.int32)
    lane = jax.lax.broadcasted_iota(jnp.int32, logits.shape, 1)
    masked = jnp.where(lane == i1, -jnp.inf, logits)
    m2 = jnp.max(masked, axis=1, keepdims=True)
    i2 = jnp.argmax(masked, axis=1)[:, None].astype(jnp.int32)
    e = jnp.exp(m2 - m1)
    w1 = 1.0 / (1.0 + e)
    idx_ref[...] = jnp.concatenate([i1, i2], axis=1)
    w_ref[...] = jnp.concatenate([w1, 1.0 - w1], axis=1)


def kernel(hidden_states, weight):
    b, s, h = hidden_states.shape
    n = b * s
    ne = weight.shape[0]
    x = hidden_states.reshape(n, h)
    wt = weight.T

    br = 512
    idx, w = pl.pallas_call(
        _gate_body,
        grid=(n // br,),
        in_specs=[
            pl.BlockSpec((br, h), lambda i: (i, 0)),
            pl.BlockSpec((h, ne), lambda i: (0, 0)),
        ],
        out_specs=[
            pl.BlockSpec((br, 2), lambda i: (i, 0)),
            pl.BlockSpec((br, 2), lambda i: (i, 0)),
        ],
        out_shape=[
            jax.ShapeDtypeStruct((n, 2), jnp.int32),
            jax.ShapeDtypeStruct((n, 2), jnp.float32),
        ],
    )(x, wt)
    aux_loss = jnp.zeros((), dtype=jnp.float32)
    return idx, w, aux_loss


# speedup vs baseline: 1.8465x; 1.1371x over previous
import jax
import jax.numpy as jnp
from jax.experimental import pallas as pl


def _gate_body(x_ref, wt_ref, idx_ref, w_ref):
    logits = jnp.dot(x_ref[...], wt_ref[...], preferred_element_type=jnp.float32)
    m1 = jnp.max(logits, axis=1, keepdims=True)
    i1 = jnp.argmax(logits, axis=1)[:, None].astype(jnp.int32)
    lane = jax.lax.broadcasted_iota(jnp.int32, logits.shape, 1)
    masked = jnp.where(lane == i1, -jnp.inf, logits)
    m2 = jnp.max(masked, axis=1, keepdims=True)
    i2 = jnp.argmax(masked, axis=1)[:, None].astype(jnp.int32)
    e = jnp.exp(m2 - m1)
    w1 = 1.0 / (1.0 + e)
    idx_ref[...] = jnp.concatenate([i1, i2], axis=1)
    w_ref[...] = jnp.concatenate([w1, 1.0 - w1], axis=1)


def kernel(hidden_states, weight):
    b, s, h = hidden_states.shape
    n = b * s
    ne = weight.shape[0]
    x = hidden_states.reshape(n, h)
    wt = weight.T

    br = 1024
    idx, w = pl.pallas_call(
        _gate_body,
        grid=(n // br,),
        in_specs=[
            pl.BlockSpec((br, h), lambda i: (i, 0)),
            pl.BlockSpec((h, ne), lambda i: (0, 0)),
        ],
        out_specs=[
            pl.BlockSpec((br, 2), lambda i: (i, 0)),
            pl.BlockSpec((br, 2), lambda i: (i, 0)),
        ],
        out_shape=[
            jax.ShapeDtypeStruct((n, 2), jnp.int32),
            jax.ShapeDtypeStruct((n, 2), jnp.float32),
        ],
    )(x, wt)
    aux_loss = jnp.zeros((), dtype=jnp.float32)
    return idx, w, aux_loss


# speedup vs baseline: 1.8478x; 1.0007x over previous
import jax
import jax.numpy as jnp
from jax.experimental import pallas as pl


def _gate_body(x_ref, wt_ref, idx_ref, w_ref):
    logits = jnp.dot(x_ref[...], wt_ref[...], preferred_element_type=jnp.float32)
    m1 = jnp.max(logits, axis=1, keepdims=True)
    i1 = jnp.argmax(logits, axis=1)[:, None].astype(jnp.int32)
    lane = jax.lax.broadcasted_iota(jnp.int32, logits.shape, 1)
    masked = jnp.where(lane == i1, -jnp.inf, logits)
    m2 = jnp.max(masked, axis=1, keepdims=True)
    i2 = jnp.argmax(masked, axis=1)[:, None].astype(jnp.int32)
    e = jnp.exp(m2 - m1)
    w1 = 1.0 / (1.0 + e)
    idx_ref[...] = jnp.concatenate([i1, i2], axis=1)
    w_ref[...] = jnp.concatenate([w1, 1.0 - w1], axis=1)


def kernel(hidden_states, weight):
    b, s, h = hidden_states.shape
    n = b * s
    ne = weight.shape[0]
    x = hidden_states.reshape(n, h)
    wt = weight.T

    br = 2048
    idx, w = pl.pallas_call(
        _gate_body,
        grid=(n // br,),
        in_specs=[
            pl.BlockSpec((br, h), lambda i: (i, 0)),
            pl.BlockSpec((h, ne), lambda i: (0, 0)),
        ],
        out_specs=[
            pl.BlockSpec((br, 2), lambda i: (i, 0)),
            pl.BlockSpec((br, 2), lambda i: (i, 0)),
        ],
        out_shape=[
            jax.ShapeDtypeStruct((n, 2), jnp.int32),
            jax.ShapeDtypeStruct((n, 2), jnp.float32),
        ],
    )(x, wt)
    aux_loss = jnp.zeros((), dtype=jnp.float32)
    return idx, w, aux_loss
